# Initial kernel scaffold; baseline (speedup 1.0000x reference)
import jax
import jax.numpy as jnp
from jax.experimental import pallas as pl


def kernel(input, weight):
    raise NotImplementedError("write your pallas kernel here")


# speedup vs baseline: 1.0938x; 1.0938x over previous
import jax
import jax.numpy as jnp
from jax import lax
from jax.experimental import pallas as pl
from jax.experimental.pallas import tpu as pltpu
from jax.experimental.pallas import tpu_sc as plsc

_DIM = 32
_B = 16384 * 50
_IW = 128
_NIDX_ROWS = _B // _IW
_NC = 2
_NS = 16
_NW = _NC * _NS
_ROWS_PER_W = _NIDX_ROWS // _NW
_CHUNK = 8
_NCHUNK = _ROWS_PER_W // _CHUNK


def _gather_body(idx_hbm, tbl_hbm, out_hbm, idx_v, rows_v, sem):
    wid = lax.axis_index("s") * _NC + lax.axis_index("c")

    def chunk_body(g, carry):
        r0 = wid * _ROWS_PER_W + g * _CHUNK
        pltpu.sync_copy(idx_hbm.at[pl.ds(r0, _CHUNK)], idx_v)
        copies = [
            pltpu.async_copy(
                tbl_hbm.at[idx_v.at[j]],
                rows_v.at[pl.ds(j * _IW, _IW)],
                sem,
            )
            for j in range(_CHUNK)
        ]
        for cp in copies:
            cp.wait()
        pltpu.sync_copy(rows_v, out_hbm.at[pl.ds(r0 * _IW, _CHUNK * _IW)])
        return carry

    lax.fori_loop(0, _NCHUNK, chunk_body, 0)


def _embed(idx2d, weight):
    k = pl.kernel(
        _gather_body,
        out_type=jax.ShapeDtypeStruct((_B, _DIM), jnp.float32),
        mesh=plsc.VectorSubcoreMesh(core_axis_name="c", subcore_axis_name="s"),
        scratch_types=[
            pltpu.VMEM((_CHUNK, _IW), jnp.int32),
            pltpu.VMEM((_CHUNK * _IW, _DIM), jnp.float32),
            pltpu.SemaphoreType.DMA,
        ],
        compiler_params=pltpu.CompilerParams(use_tc_tiling_on_sc=False),
    )
    return k(idx2d, weight)


def kernel(input, weight):
    idx2d = input.astype(jnp.int32).reshape(_NIDX_ROWS, _IW)
    out = _embed(idx2d, weight)
    return out.reshape(input.shape + (weight.shape[1],))


# speedup vs baseline: 1.1078x; 1.0128x over previous
import jax
import jax.numpy as jnp
from jax import lax
from jax.experimental import pallas as pl
from jax.experimental.pallas import tpu as pltpu
from jax.experimental.pallas import tpu_sc as plsc

_DIM = 32
_B = 16384 * 50
_IW = 128
_NIDX_ROWS = _B // _IW
_NC = 2
_NS = 16
_NW = _NC * _NS
_ROWS_PER_W = _NIDX_ROWS // _NW
_CHUNK = 8
_NCHUNK = _ROWS_PER_W // _CHUNK


_NBUF = 3


def _gather_body(idx_hbm, tbl_hbm, out_hbm, idx_v, rows_v, sem_i, sem_g, sem_o):
    wid = lax.axis_index("s") * _NC + lax.axis_index("c")
    row0 = wid * _ROWS_PER_W

    def start_idx(g):
        return pltpu.async_copy(
            idx_hbm.at[pl.ds(row0 + g * _CHUNK, _CHUNK)],
            idx_v.at[g % _NBUF], sem_i)

    def start_gather(g):
        b = g % _NBUF
        return [
            pltpu.async_copy(
                tbl_hbm.at[idx_v.at[b].at[j]],
                rows_v.at[b].at[pl.ds(j * _IW, _IW)], sem_g)
            for j in range(_CHUNK)
        ]

    def start_out(g):
        return pltpu.async_copy(
            rows_v.at[g % _NBUF],
            out_hbm.at[pl.ds((row0 + g * _CHUNK) * _IW, _CHUNK * _IW)], sem_o)

    idxc, outc = {}, {}
    idxc[0] = start_idx(0)
    for g in range(_NCHUNK):
        idxc[g].wait()
        if g + 1 < _NCHUNK:
            idxc[g + 1] = start_idx(g + 1)
        if g >= _NBUF:
            outc[g - _NBUF].wait()
        for cp in start_gather(g):
            cp.wait()
        outc[g] = start_out(g)
    for g in range(max(0, _NCHUNK - _NBUF), _NCHUNK):
        outc[g].wait()


def _embed(idx2d, weight):
    k = pl.kernel(
        _gather_body,
        out_type=jax.ShapeDtypeStruct((_B, _DIM), jnp.float32),
        mesh=plsc.VectorSubcoreMesh(core_axis_name="c", subcore_axis_name="s"),
        scratch_types=[
            pltpu.VMEM((_NBUF, _CHUNK, _IW), jnp.int32),
            pltpu.VMEM((_NBUF, _CHUNK * _IW, _DIM), jnp.float32),
            pltpu.SemaphoreType.DMA,
            pltpu.SemaphoreType.DMA,
            pltpu.SemaphoreType.DMA,
        ],
        compiler_params=pltpu.CompilerParams(use_tc_tiling_on_sc=False),
    )
    return k(idx2d, weight)


def kernel(input, weight):
    idx2d = input.astype(jnp.int32).reshape(_NIDX_ROWS, _IW)
    out = _embed(idx2d, weight)
    return out.reshape(input.shape + (weight.shape[1],))


# speedup vs baseline: 1.7348x; 1.5660x over previous
import jax
import jax.numpy as jnp
from jax import lax
from jax.experimental import pallas as pl
from jax.experimental.pallas import tpu as pltpu
from jax.experimental.pallas import tpu_sc as plsc

_DIM = 32
_BATCH = 16384
_HIST = 50
_IW = 128
_NC = 2
_NS = 16
_NW = _NC * _NS
_CPW = (_BATCH // _IW) // _NW
_D4 = _DIM // 8


def _body(idx_hbm, tbl_hbm, out_hbm, idx_v, g_v, t_v, sem_i, sem_g, sem_o):
    wid = lax.axis_index("s") * _NC + lax.axis_index("c")
    b0 = wid * (_CPW * _IW)
    c0 = wid * _CPW

    lane = lax.iota(jnp.int32, 16)
    d4v = [(lane >> 3) + 2 * dg for dg in range(2)]
    sv = lane & 7

    def idx_copy(h, slot):
        return pltpu.make_async_copy(
            idx_hbm.at[h, pl.ds(b0, _CPW * _IW)], idx_v.at[slot], sem_i)

    def out_copy(h, slot, j):
        return pltpu.make_async_copy(
            t_v.at[slot].at[j], out_hbm.at[h, :, c0 + j], sem_o)

    def transpose_item(slot, j):
        gref = g_v.at[slot].at[j]
        tref = t_v.at[slot].at[j]

        def tb(b, carry):
            bv = jnp.full((16,), b, jnp.int32)
            for dg in range(2):
                v = gref[b, pl.ds(dg * 16, 16)]
                plsc.store_scatter(tref, [d4v[dg], sv, bv], v)
            return carry

        lax.fori_loop(0, _IW, tb, 0, unroll=8)

    idx_copy(0, 0).start()
    idx_copy(1, 1).start()
    for slot in range(2):
        for j in range(_CPW):
            out_copy(slot, slot, j).start()

    def step(i, carry):
        for slot in range(2):
            h = 2 * i + slot
            idx_copy(h, slot).wait()
            gc = [
                pltpu.make_async_copy(
                    tbl_hbm.at[idx_v.at[slot].at[pl.ds(j * _IW, _IW)]],
                    g_v.at[slot].at[j], sem_g)
                for j in range(_CPW)
            ]
            for c in gc:
                c.start()
            for c in gc:
                c.wait()
            idx_copy(jnp.minimum(h + 2, _HIST - 1), slot).start()
            for j in range(_CPW):
                out_copy(h, slot, j).wait()
                transpose_item(slot, j)
                out_copy(h, slot, j).start()
        return carry

    lax.fori_loop(0, _HIST // 2, step, 0)

    for slot in range(2):
        idx_copy(_HIST - 1, slot).wait()
        for j in range(_CPW):
            out_copy(_HIST - 2 + slot, slot, j).wait()


def _embed(idx_t, weight):
    k = pl.kernel(
        _body,
        out_type=jax.ShapeDtypeStruct((_HIST, _D4, _BATCH // _IW, 8, _IW),
                                      jnp.float32),
        mesh=plsc.VectorSubcoreMesh(core_axis_name="c", subcore_axis_name="s"),
        scratch_types=[
            pltpu.VMEM((2, _CPW * _IW), jnp.int32),
            pltpu.VMEM((2, _CPW, _IW, _DIM), jnp.float32),
            pltpu.VMEM((2, _CPW, _D4, 8, _IW), jnp.float32),
            pltpu.SemaphoreType.DMA,
            pltpu.SemaphoreType.DMA,
            pltpu.SemaphoreType.DMA,
        ],
        compiler_params=pltpu.CompilerParams(use_tc_tiling_on_sc=False,
                                             needs_layout_passes=False),
    )
    return k(idx_t, weight)


def kernel(input, weight):
    idx_t = input.astype(jnp.int32).T
    out5 = _embed(idx_t, weight)
    out = out5.transpose(2, 4, 0, 1, 3).reshape(_BATCH, _HIST, _DIM)
    return out


# speedup vs baseline: 1.9961x; 1.1506x over previous
import jax
import jax.numpy as jnp
from jax import lax
from jax.experimental import pallas as pl
from jax.experimental.pallas import tpu as pltpu
from jax.experimental.pallas import tpu_sc as plsc

_DIM = 32
_BATCH = 16384
_HIST = 50
_IW = 128
_NC = 2
_NS = 16
_NW = _NC * _NS
_CPW = (_BATCH // _IW) // _NW
_D4 = _DIM // 8


def _body(idx_hbm, tbl_hbm, out_hbm, idx_v, g_v, t_v, sem_i, sem_g, sem_o):
    wid = lax.axis_index("s") * _NC + lax.axis_index("c")
    b0 = wid * (_CPW * _IW)
    c0 = wid * _CPW

    lane = lax.iota(jnp.int32, 16)
    d4v = [(lane >> 3) + 2 * dg for dg in range(2)]
    sv = lane & 7

    def idx_copy(h, slot):
        return pltpu.make_async_copy(
            idx_hbm.at[h, pl.ds(b0, _CPW * _IW)], idx_v.at[slot], sem_i)

    def out_copy(h, slot, j):
        return pltpu.make_async_copy(
            t_v.at[slot].at[j], out_hbm.at[h, :, c0 + j], sem_o)

    def transpose_item(slot, j):
        gref = g_v.at[slot].at[j]
        tref = t_v.at[slot].at[j]

        @plsc.parallel_loop(0, _IW, 1, unroll=8)
        def _tb(b):
            bv = jnp.full((16,), b, jnp.int32)
            for dg in range(2):
                v = gref[b, pl.ds(dg * 16, 16)]
                plsc.store_scatter(tref, [d4v[dg], sv, bv], v)

    idx_copy(0, 0).start()
    idx_copy(1, 1).start()
    for slot in range(2):
        for j in range(_CPW):
            out_copy(slot, slot, j).start()

    def step(i, carry):
        for slot in range(2):
            h = 2 * i + slot
            idx_copy(h, slot).wait()
            gc = [
                pltpu.make_async_copy(
                    tbl_hbm.at[idx_v.at[slot].at[pl.ds(j * _IW, _IW)]],
                    g_v.at[slot].at[j], sem_g)
                for j in range(_CPW)
            ]
            for c in gc:
                c.start()
            for c in gc:
                c.wait()
            idx_copy(jnp.minimum(h + 2, _HIST - 1), slot).start()
            for j in range(_CPW):
                out_copy(h, slot, j).wait()
                transpose_item(slot, j)
                out_copy(h, slot, j).start()
        return carry

    lax.fori_loop(0, _HIST // 2, step, 0)

    for slot in range(2):
        idx_copy(_HIST - 1, slot).wait()
        for j in range(_CPW):
            out_copy(_HIST - 2 + slot, slot, j).wait()


def _embed(idx_t, weight):
    k = pl.kernel(
        _body,
        out_type=jax.ShapeDtypeStruct((_HIST, _D4, _BATCH // _IW, 8, _IW),
                                      jnp.float32),
        mesh=plsc.VectorSubcoreMesh(core_axis_name="c", subcore_axis_name="s"),
        scratch_types=[
            pltpu.VMEM((2, _CPW * _IW), jnp.int32),
            pltpu.VMEM((2, _CPW, _IW, _DIM), jnp.float32),
            pltpu.VMEM((2, _CPW, _D4, 8, _IW), jnp.float32),
            pltpu.SemaphoreType.DMA,
            pltpu.SemaphoreType.DMA,
            pltpu.SemaphoreType.DMA,
        ],
        compiler_params=pltpu.CompilerParams(use_tc_tiling_on_sc=False,
                                             needs_layout_passes=False),
    )
    return k(idx_t, weight)


def kernel(input, weight):
    idx_t = input.astype(jnp.int32).T
    out5 = _embed(idx_t, weight)
    out = out5.transpose(2, 4, 0, 1, 3).reshape(_BATCH, _HIST, _DIM)
    return out
